# baseline (speedup 1.0000x reference)
import functools

import jax
import jax.numpy as jnp
from jax import lax
from jax.experimental import pallas as pl
from jax.experimental.pallas import tpu as pltpu
from jax.experimental.pallas import tpu_sc as plsc

_N = 1000000
_D = 64
_B = 16384
_W = 512
_LANES = 16
_REG_END = (_N // _W) * _W
_TW = _N - _REG_END


def _make_gather():
    info = plsc.get_sparse_core_info()
    nc, ns = info.num_cores, info.num_subcores
    nw = nc * ns
    rng = 32768
    mesh = plsc.VectorSubcoreMesh(core_axis_name="c", subcore_axis_name="s")

    @functools.partial(
        pl.kernel,
        mesh=mesh,
        out_type=jax.ShapeDtypeStruct((_B + _LANES, 128), jnp.float32),
        scratch_types=[
            pltpu.VMEM((_B,), jnp.int32),
            pltpu.VMEM((_B + _LANES,), jnp.int32),
            pltpu.VMEM((_B + _LANES,), jnp.int32),
            pltpu.VMEM((_D, _W), jnp.float32),
            pltpu.VMEM((_D, _W), jnp.float32),
            pltpu.VMEM((_LANES, 128), jnp.float32),
            pltpu.VMEM((_D, _TW), jnp.float32),
            pltpu.SemaphoreType.DMA,
            pltpu.SemaphoreType.DMA,
            pltpu.SemaphoreType.DMA,
        ],
        compiler_params=pltpu.CompilerParams(needs_layout_passes=False),
    )
    def gather_kernel(idx_hbm, tabt_hbm, tail_hbm, out_hbm, idx_v, own_i,
                      own_j, buf0, buf1, rowbuf, tailbuf, sem0, sem1,
                      out_sem):
        wid = lax.axis_index("s") * nc + lax.axis_index("c")
        lane = lax.iota(jnp.int32, _LANES)
        bufs = (buf0, buf1)
        sems = (sem0, sem1)

        lo = jnp.minimum(wid * rng, _REG_END)
        hi = jnp.minimum(lo + rng, _REG_END)
        hic = jnp.where(wid == nw - 1, _N, hi)
        nch = ((hi - lo) // _W + 1) // 2 * 2

        pltpu.sync_copy(idx_hbm, idx_v)

        big = jnp.int32(0x7F000000)

        def collect(g, n):
            iv = idx_v[pl.ds(g * _LANES, _LANES)]
            m = (iv >= lo) & (iv < hic)
            cnt = plsc.all_reduce_population_count(m)[0]
            own_i[pl.ds(n, _LANES)] = jnp.where(m, iv, big)
            own_j[pl.ds(n, _LANES)] = lane + g * _LANES
            return n + jnp.where(cnt > 0, _LANES, 0).astype(jnp.int32)

        n_own = lax.fori_loop(0, _B // _LANES, collect, jnp.int32(0))
        ngrp = n_own // _LANES

        def drain_one(_, c):
            pltpu.make_async_copy(out_hbm.at[pl.ds(0, _LANES)],
                                  rowbuf, out_sem).wait()
            return c

        def serve(buf, i0, w, kd):
            def per_group(g, kd):
                kk, dd = kd
                iv = own_i[pl.ds(g * _LANES, _LANES)]
                rel = iv - i0
                m = (rel >= 0) & (rel < w)
                ng = plsc.all_reduce_population_count(m)[0]

                @pl.when(ng > 0)
                def _():
                    lax.fori_loop(0, kk - dd, drain_one, 0)
                    jv = own_j[pl.ds(g * _LANES, _LANES)]
                    jfull = jnp.where(m, jv, jnp.int32(_B))
                    for c in range(_D):
                        cv = jnp.full((_LANES,), c, jnp.int32)
                        xc = plsc.load_gather(buf, [cv, rel], mask=m)
                        plsc.store_scatter(rowbuf, [lane, cv], xc, mask=m)
                    pltpu.async_copy(rowbuf, out_hbm.at[jfull], out_sem)

                hit = jnp.where(ng > 0, 1, 0).astype(jnp.int32)
                return (kk + hit, jnp.where(ng > 0, kk, dd))

            return lax.fori_loop(0, ngrp, per_group, kd)

        def chunk_off(c):
            return jnp.minimum(lo + c * _W, _REG_END - _W)

        def start(c, b):
            i0 = pl.multiple_of(chunk_off(c), 128)
            for r8 in range(_D // 8):
                pltpu.async_copy(
                    tabt_hbm.at[pl.ds(r8 * 8, 8), pl.ds(i0, _W)],
                    bufs[b].at[pl.ds(r8 * 8, 8)], sems[b])

        def waitb(b):
            for r8 in range(_D // 8):
                pltpu.make_async_copy(
                    tabt_hbm.at[pl.ds(0, 8), pl.ds(0, _W)],
                    bufs[b].at[pl.ds(r8 * 8, 8)], sems[b]).wait()

        @pl.when(nch > 0)
        def _():
            start(0, 0)

        def pair(p, kd):
            for b in range(2):
                cc = 2 * p + b
                waitb(b)

                @pl.when(cc + 1 < nch)
                def _():
                    start(cc + 1, 1 - b)

                kd = serve(bufs[b], chunk_off(cc), _W, kd)
            return kd

        kd = lax.fori_loop(0, nch // 2, pair,
                           (jnp.int32(0), jnp.int32(0)))

        pltpu.sync_copy(tail_hbm, tailbuf)
        kd = serve(tailbuf, jnp.int32(_REG_END), _TW, kd)

        kk, dd = kd
        lax.fori_loop(0, kk - dd, drain_one, 0)

    return gather_kernel


def kernel(user_idx, table):
    tabt = table.T
    tail = tabt[:, _REG_END:]
    out = _make_gather()(user_idx.astype(jnp.int32), tabt, tail)
    return out[:_B, :_D]


# speedup vs baseline: 33.6740x; 33.6740x over previous
import functools

import jax
import jax.numpy as jnp
from jax import lax
from jax.experimental import pallas as pl
from jax.experimental.pallas import tpu as pltpu
from jax.experimental.pallas import tpu_sc as plsc

_NUM_USERS = 1000000
_EMBED_DIM = 64
_BATCH = 16384
_RPB = 8
_CH = 32
_LANES = 16


def _make_gather(batch, dim):
    info = plsc.get_sparse_core_info()
    nc, ns = info.num_cores, info.num_subcores
    nw = nc * ns
    b_per_w = batch // nw
    nch = b_per_w // _CH
    obpc = _CH // _RPB
    mesh = plsc.VectorSubcoreMesh(core_axis_name="c", subcore_axis_name="s")

    @functools.partial(
        pl.kernel,
        mesh=mesh,
        out_type=jax.ShapeDtypeStruct((batch // _RPB, _RPB, dim),
                                      jnp.float32),
        scratch_types=[
            pltpu.VMEM((b_per_w,), jnp.int32),
            pltpu.VMEM((b_per_w,), jnp.int32),
            pltpu.VMEM((_CH, _RPB, dim), jnp.float32),
            pltpu.VMEM((_CH, _RPB, dim), jnp.float32),
            pltpu.VMEM((obpc, _RPB, dim), jnp.float32),
            pltpu.SemaphoreType.DMA,
            pltpu.SemaphoreType.DMA,
        ],
        compiler_params=pltpu.CompilerParams(needs_layout_passes=False),
    )
    def gather_kernel(idx_hbm, tab_hbm, out_hbm, idx_v, blk_v, buf0, buf1,
                      out_v, sem0, sem1):
        wid = lax.axis_index("s") * nc + lax.axis_index("c")
        base = wid * b_per_w
        bufs = (buf0, buf1)
        sems = (sem0, sem1)

        pltpu.sync_copy(idx_hbm.at[pl.ds(base, b_per_w)], idx_v)
        for s in range(b_per_w // _LANES):
            sl = pl.ds(s * _LANES, _LANES)
            blk_v[sl] = idx_v[sl] >> 3

        lane = lax.iota(jnp.int32, _LANES)

        def start(g, b):
            for h in range(_CH // _LANES):
                blks = blk_v[pl.ds(g * _CH + h * _LANES, _LANES)]
                for l in range(_LANES):
                    s = jnp.sum(jnp.where(lane == l, blks, 0))
                    j = h * _LANES + l
                    pltpu.async_copy(tab_hbm.at[s], bufs[b].at[j], sems[b])

        def extract(g, b):
            lane = lax.iota(jnp.int32, _LANES)
            for h in range(_CH // _LANES):
                rems = idx_v[pl.ds(g * _CH + h * _LANES, _LANES)] & 7
                src0 = lane + h * _LANES
                jj = src0
                dst0 = jj >> 3
                dst1 = jj & 7
                for c in range(dim):
                    col = jnp.full((_LANES,), c, jnp.int32)
                    x = plsc.load_gather(bufs[b], [src0, rems, col])
                    plsc.store_scatter(out_v, [dst0, dst1, col], x)
            pltpu.sync_copy(
                out_v, out_hbm.at[pl.ds(wid * (b_per_w // _RPB) + g * obpc,
                                        obpc)])

        start(0, 0)

        def step(i, carry):
            g = i * 2
            for b in range(2):
                gg = g + b
                pltpu.make_async_copy(
                    tab_hbm.at[pl.ds(0, _CH)], bufs[b], sems[b]).wait()

                @pl.when(gg + 1 < nch)
                def _():
                    start(gg + 1, 1 - b)

                extract(gg, b)
            return carry

        lax.fori_loop(0, nch // 2, step, 0)

    return gather_kernel


def kernel(user_idx, table):
    tab3 = table.reshape(_NUM_USERS // _RPB, _RPB, _EMBED_DIM)
    out3 = _make_gather(_BATCH, _EMBED_DIM)(user_idx.astype(jnp.int32), tab3)
    return out3.reshape(_BATCH, _EMBED_DIM)


# speedup vs baseline: 34.0485x; 1.0111x over previous
import functools

import jax
import jax.numpy as jnp
from jax import lax
from jax.experimental import pallas as pl
from jax.experimental.pallas import tpu as pltpu
from jax.experimental.pallas import tpu_sc as plsc

_NUM_USERS = 1000000
_EMBED_DIM = 64
_BATCH = 16384
_RPB = 8
_CH = 32
_LANES = 16


def _make_gather(batch, dim):
    info = plsc.get_sparse_core_info()
    nc, ns = info.num_cores, info.num_subcores
    nw = nc * ns
    b_per_w = batch // nw
    nch = b_per_w // _CH
    obpc = _CH // _RPB
    mesh = plsc.VectorSubcoreMesh(core_axis_name="c", subcore_axis_name="s")

    @functools.partial(
        pl.kernel,
        mesh=mesh,
        out_type=jax.ShapeDtypeStruct((batch // _RPB, _RPB, dim),
                                      jnp.float32),
        scratch_types=[
            pltpu.VMEM((b_per_w,), jnp.int32),
            pltpu.VMEM((b_per_w,), jnp.int32),
            pltpu.VMEM((_CH, _RPB, dim), jnp.float32),
            pltpu.VMEM((_CH, _RPB, dim), jnp.float32),
            pltpu.VMEM((obpc, _RPB, dim), jnp.float32),
            pltpu.SemaphoreType.DMA,
            pltpu.SemaphoreType.DMA,
        ],
        compiler_params=pltpu.CompilerParams(needs_layout_passes=False),
    )
    def gather_kernel(idx_hbm, tab_hbm, out_hbm, idx_v, blk_v, buf0, buf1,
                      out_v, sem0, sem1):
        wid = lax.axis_index("s") * nc + lax.axis_index("c")
        base = wid * b_per_w
        bufs = (buf0, buf1)
        sems = (sem0, sem1)

        pltpu.sync_copy(idx_hbm.at[pl.ds(base, b_per_w)], idx_v)
        for s in range(b_per_w // _LANES):
            sl = pl.ds(s * _LANES, _LANES)
            blk_v[sl] = idx_v[sl] >> 3

        lane = lax.iota(jnp.int32, _LANES)

        def start(g, b):
            for h in range(_CH // _LANES):
                blks = blk_v[pl.ds(g * _CH + h * _LANES, _LANES)]
                for l in range(_LANES):
                    s = blks[l]
                    j = h * _LANES + l
                    pltpu.async_copy(tab_hbm.at[s], bufs[b].at[j], sems[b])

        def extract(g, b):
            lane = lax.iota(jnp.int32, _LANES)
            for h in range(_CH // _LANES):
                rems = idx_v[pl.ds(g * _CH + h * _LANES, _LANES)] & 7
                src0 = lane + h * _LANES
                jj = src0
                dst0 = jj >> 3
                dst1 = jj & 7
                for c in range(dim):
                    col = jnp.full((_LANES,), c, jnp.int32)
                    x = plsc.load_gather(bufs[b], [src0, rems, col])
                    plsc.store_scatter(out_v, [dst0, dst1, col], x)
            pltpu.sync_copy(
                out_v, out_hbm.at[pl.ds(wid * (b_per_w // _RPB) + g * obpc,
                                        obpc)])

        start(0, 0)

        def step(i, carry):
            g = i * 2
            for b in range(2):
                gg = g + b
                pltpu.make_async_copy(
                    tab_hbm.at[pl.ds(0, _CH)], bufs[b], sems[b]).wait()

                @pl.when(gg + 1 < nch)
                def _():
                    start(gg + 1, 1 - b)

                extract(gg, b)
            return carry

        lax.fori_loop(0, nch // 2, step, 0)

    return gather_kernel


def kernel(user_idx, table):
    tab3 = table.reshape(_NUM_USERS // _RPB, _RPB, _EMBED_DIM)
    out3 = _make_gather(_BATCH, _EMBED_DIM)(user_idx.astype(jnp.int32), tab3)
    return out3.reshape(_BATCH, _EMBED_DIM)
